# baseline (speedup 1.0000x reference)
import jax
import jax.numpy as jnp
from jax.experimental import pallas as pl


def _pos_kernel(row_ref, col_ref, out_ref):
    h = out_ref.shape[2]
    w = out_ref.shape[3]
    ce = col_ref[:w, :]
    re = row_ref[:h, :]
    top = jnp.transpose(ce)[:, None, :]
    bot = jnp.transpose(re)[:, :, None]
    top = jnp.broadcast_to(top, (top.shape[0], h, w))
    bot = jnp.broadcast_to(bot, (bot.shape[0], h, w))
    out_ref[0] = jnp.concatenate([top, bot], axis=0)


def kernel(x, row_embed, col_embed):
    b = x.shape[0]
    h, w = x.shape[-2], x.shape[-1]
    d = row_embed.shape[-1]
    out_shape = jax.ShapeDtypeStruct((b, 2 * d, h, w), row_embed.dtype)
    return pl.pallas_call(
        _pos_kernel,
        grid=(b,),
        in_specs=[
            pl.BlockSpec(row_embed.shape, lambda i: (0, 0)),
            pl.BlockSpec(col_embed.shape, lambda i: (0, 0)),
        ],
        out_specs=pl.BlockSpec((1, 2 * d, h, w), lambda i: (i, 0, 0, 0)),
        out_shape=out_shape,
    )(row_embed, col_embed)


# speedup vs baseline: 3.0361x; 3.0361x over previous
import jax
import jax.numpy as jnp
from jax.experimental import pallas as pl
from jax.experimental.pallas import tpu as pltpu


def _pos_kernel(row_ref, col_ref, out_ref, scratch, sems):
    b, two_d, hw = out_ref.shape
    d = two_d // 2
    h = row_ref.shape[0]
    w = col_ref.shape[0]
    ceT = jnp.transpose(col_ref[:, :])
    reT = jnp.transpose(row_ref[:, :])
    top = jnp.broadcast_to(ceT[:, None, :], (d, h, w)).reshape(d, hw)
    bot = jnp.broadcast_to(reT[:, :, None], (d, h, w)).reshape(d, hw)
    scratch[:d] = top
    scratch[d:] = bot
    for i in range(b):
        pltpu.make_async_copy(scratch, out_ref.at[i], sems.at[i]).start()
    for i in range(b):
        pltpu.make_async_copy(scratch, out_ref.at[i], sems.at[i]).wait()


def kernel(x, row_embed, col_embed):
    b = x.shape[0]
    h, w = x.shape[-2], x.shape[-1]
    d = row_embed.shape[-1]
    out = pl.pallas_call(
        _pos_kernel,
        in_specs=[
            pl.BlockSpec((h, d), lambda: (0, 0)),
            pl.BlockSpec((w, d), lambda: (0, 0)),
        ],
        out_specs=pl.BlockSpec(memory_space=pl.ANY),
        out_shape=jax.ShapeDtypeStruct((b, 2 * d, h * w), row_embed.dtype),
        scratch_shapes=[
            pltpu.VMEM((2 * d, h * w), row_embed.dtype),
            pltpu.SemaphoreType.DMA((b,)),
        ],
    )(row_embed[:h], col_embed[:w])
    return out.reshape(b, 2 * d, h, w)


# speedup vs baseline: 3.0504x; 1.0047x over previous
import jax
import jax.numpy as jnp
from jax.experimental import pallas as pl
from jax.experimental.pallas import tpu as pltpu


def _pos_kernel(row_ref, col_ref, out_ref, scratch):
    _, two_d, hw = out_ref.shape
    d = two_d // 2
    h = row_ref.shape[0]
    w = col_ref.shape[0]

    @pl.when(pl.program_id(0) == 0)
    def _():
        ceT = jnp.transpose(col_ref[:, :])
        reT = jnp.transpose(row_ref[:, :])
        scratch[:d] = jnp.broadcast_to(ceT[:, None, :], (d, h, w)).reshape(d, hw)
        scratch[d:] = jnp.broadcast_to(reT[:, :, None], (d, h, w)).reshape(d, hw)

    out_ref[0] = scratch[:, :]


def kernel(x, row_embed, col_embed):
    b = x.shape[0]
    h, w = x.shape[-2], x.shape[-1]
    d = row_embed.shape[-1]
    out = pl.pallas_call(
        _pos_kernel,
        grid=(b,),
        in_specs=[
            pl.BlockSpec((h, d), lambda i: (0, 0)),
            pl.BlockSpec((w, d), lambda i: (0, 0)),
        ],
        out_specs=pl.BlockSpec((1, 2 * d, h * w), lambda i: (i, 0, 0)),
        out_shape=jax.ShapeDtypeStruct((b, 2 * d, h * w), row_embed.dtype),
        scratch_shapes=[pltpu.VMEM((2 * d, h * w), row_embed.dtype)],
    )(row_embed[:h], col_embed[:w])
    return out.reshape(b, 2 * d, h, w)
